# baseline (speedup 1.0000x reference)
import functools

import jax
import jax.numpy as jnp
from jax import lax
from jax.experimental import pallas as pl
from jax.experimental.pallas import tpu as pltpu
from jax.experimental.pallas import tpu_sc as plsc

N = 10000
NPAD = 10240
IN = 128
OUT = 32
D4 = 4
RK = 10
DK = 16
NEG = 0.2

NC = 2
NS = 16
NW = NC * NS
NT = NPAD // NW
NG = NT // 16
DCH = 32
NDCH = NT // DCH
RLEN = D4 * NT
NQ = OUT // 4
QI = NQ // 2


def _leaky(x):
    return jnp.where(x >= 0, x, NEG * x)


def _sigmoid(x):
    return 1.0 / (1.0 + jnp.exp(-x))


BN = 512


def _tc_body(h_ref, wc_ref, bm_ref, whT_ref, whdT_ref, st_ref):
    PT = lax.dot_general(wc_ref[...], h_ref[...], (((0,), (1,)), ((), ())),
                         preferred_element_type=jnp.float32)
    whT_ref[...] = PT[:OUT]
    whdT_ref[...] = PT[OUT:]
    st_ref[...] = lax.dot_general(bm_ref[...], PT, (((0,), (0,)), ((), ())),
                                  preferred_element_type=jnp.float32)


def _tc_project(h_pad, Wc, Bmat):
    return pl.pallas_call(
        _tc_body,
        grid=(NPAD // BN,),
        in_specs=[
            pl.BlockSpec((BN, IN), lambda i: (i, 0)),
            pl.BlockSpec((IN, 2 * OUT), lambda i: (0, 0)),
            pl.BlockSpec((2 * OUT, 8), lambda i: (0, 0)),
        ],
        out_specs=[
            pl.BlockSpec((OUT, BN), lambda i: (0, i)),
            pl.BlockSpec((OUT, BN), lambda i: (0, i)),
            pl.BlockSpec((8, BN), lambda i: (0, i)),
        ],
        out_shape=[
            jax.ShapeDtypeStruct((OUT, NPAD), jnp.float32),
            jax.ShapeDtypeStruct((OUT, NPAD), jnp.float32),
            jax.ShapeDtypeStruct((8, NPAD), jnp.float32),
        ],
    )(h_pad, Wc, Bmat)


def _make_sc_kernel():
    mesh = plsc.VectorSubcoreMesh(core_axis_name="c", subcore_axis_name="s",
                                  num_cores=NC, num_subcores=NS)
    scratch = [
        pltpu.VMEM((NPAD,), jnp.int32),
        pltpu.VMEM((NT,), jnp.float32),
        pltpu.VMEM((NT,), jnp.float32),
        pltpu.VMEM((NT * D4 * RK,), jnp.int32),
        pltpu.VMEM((NT * DK,), jnp.int32),
        pltpu.VMEM((2 * NPAD,), jnp.int32),
        pltpu.VMEM((2 * NPAD,), jnp.int32),
        pltpu.VMEM((2 * NPAD,), jnp.int32),
        pltpu.VMEM((2 * NPAD,), jnp.int32),
        pltpu.VMEM((D4, NT), jnp.float32),
        pltpu.VMEM((DK, NT), jnp.float32),
        pltpu.VMEM((4 * NT,), jnp.float32),
        pltpu.SemaphoreType.DMA,
        pltpu.SemaphoreType.DMA,
    ]

    @functools.partial(
        pl.kernel,
        out_type=jax.ShapeDtypeStruct((OUT * NPAD,), jnp.float32),
        mesh=mesh,
        scratch_types=scratch,
        compiler_params=pltpu.CompilerParams(needs_layout_passes=False,
                                             use_tc_tiling_on_sc=False),
    )
    def sc_kernel(whp_hbm, wdp_hbm, st_hbm, tpk_hbm, ridx_hbm, didx_hbm,
                  out_hbm, tpk_v, s1_v, s2_v, ridx_v, didx_v,
                  qwA, qdA, qwB, qdB, alpha_v, ad_v, outq_v,
                  sem_a, sem_b):
        sid = lax.axis_index("s")
        wid = sid * NC + lax.axis_index("c")
        base = wid * NT

        def _unlo(w):
            return plsc.bitcast(w << 16, jnp.float32)

        def _unhi(w):
            return plsc.bitcast(w & jnp.int32(-65536), jnp.float32)

        pltpu.sync_copy(tpk_hbm, tpk_v)
        pltpu.sync_copy(st_hbm.at[pl.ds(base, NT)], s1_v)
        pltpu.sync_copy(st_hbm.at[pl.ds(2 * NPAD + base, NT)], s2_v)
        pltpu.sync_copy(ridx_hbm.at[pl.ds(wid * (NT * D4 * RK), NT * D4 * RK)],
                        ridx_v)
        pltpu.sync_copy(didx_hbm.at[pl.ds(wid * (NT * DK), NT * DK)], didx_v)

        pre = [pltpu.async_copy(whp_hbm.at[pl.ds(0, 2 * NPAD)], qwA, sem_a),
               pltpu.async_copy(wdp_hbm.at[pl.ds(0, 2 * NPAD)], qdA, sem_a),
               pltpu.async_copy(whp_hbm.at[pl.ds(2 * NPAD, 2 * NPAD)], qwB,
                                sem_b),
               pltpu.async_copy(wdp_hbm.at[pl.ds(2 * NPAD, 2 * NPAD)], qdB,
                                sem_b)]

        def attn_group(g, carry):
            goff = g * 16
            nloc = goff + lax.iota(jnp.int32, 16)
            nvR = nloc * (D4 * RK)
            nvD = nloc * DK
            s1 = s1_v[pl.ds(goff, 16)]
            es = []
            for d in range(D4):
                acc = None
                for k in range(RK):
                    iv = plsc.load_gather(ridx_v, [nvR + (d * RK + k)])
                    t = _unlo(plsc.load_gather(tpk_v, [iv]))
                    acc = t if acc is None else acc + t
                es.append(_leaky(s1 + (1.0 / RK) * acc))
            m = jnp.maximum(jnp.maximum(es[0], es[1]),
                            jnp.maximum(es[2], es[3]))
            ex = [jnp.exp(e - m) for e in es]
            inv = (1.0 / RK) / ((ex[0] + ex[1]) + (ex[2] + ex[3]))
            for d in range(D4):
                alpha_v[d, pl.ds(goff, 16)] = ex[d] * inv

            s2 = s2_v[pl.ds(goff, 16)]
            eds = []
            for k in range(DK):
                iv = plsc.load_gather(didx_v, [nvD + k])
                eds.append(_leaky(s2 + _unhi(plsc.load_gather(tpk_v, [iv]))))
            m2 = functools.reduce(jnp.maximum, eds)
            ex2 = [jnp.exp(e - m2) for e in eds]
            inv2 = 1.0 / functools.reduce(lambda a, b: a + b, ex2)
            for k in range(DK):
                ad_v[k, pl.ds(goff, 16)] = ex2[k] * inv2
            return carry

        lax.fori_loop(0, NG, attn_group, 0)

        def make_quad_compute(col_wh, col_wd):
            def gbody(g, carry):
                goff = g * 16
                nloc = goff + lax.iota(jnp.int32, 16)
                nvR = nloc * (D4 * RK)
                nvD = nloc * DK
                als = [alpha_v[d, pl.ds(goff, 16)] for d in range(D4)]
                accR = [None] * 4
                for d in range(D4):
                    ts = [None] * 4
                    for k in range(RK):
                        iv = plsc.load_gather(ridx_v, [nvR + (d * RK + k)])
                        w0 = plsc.load_gather(col_wh, [iv])
                        w1 = plsc.load_gather(col_wh, [iv + NPAD])
                        for j, part in enumerate(
                                (_unlo(w0), _unhi(w0), _unlo(w1), _unhi(w1))):
                            ts[j] = part if ts[j] is None else ts[j] + part
                    for j in range(4):
                        wv = als[d] * ts[j]
                        accR[j] = wv if accR[j] is None else accR[j] + wv
                accD = [None] * 4
                for k in range(DK):
                    iv = plsc.load_gather(didx_v, [nvD + k])
                    w0 = plsc.load_gather(col_wd, [iv])
                    w1 = plsc.load_gather(col_wd, [iv + NPAD])
                    adk = ad_v[k, pl.ds(goff, 16)]
                    for j, part in enumerate(
                            (_unlo(w0), _unhi(w0), _unlo(w1), _unhi(w1))):
                        wv = adk * part
                        accD[j] = wv if accD[j] is None else accD[j] + wv
                for j in range(4):
                    val = 0.5 * (_sigmoid(accR[j]) + _sigmoid(accD[j]))
                    plsc.store_scatter(outq_v, [j * NT + nloc], val)
                return carry

            return gbody

        gbody_A = make_quad_compute(qwA, qdA)
        gbody_B = make_quad_compute(qwB, qdB)

        def _flush(q):
            for j in range(4):
                pltpu.sync_copy(
                    outq_v.at[pl.ds(j * NT, NT)],
                    out_hbm.at[pl.ds((4 * q + j) * NPAD + base, NT)])

        def pair_body(p, carry):
            qA = 2 * p
            qB = 2 * p + 1
            pre[0].wait()
            pre[1].wait()
            lax.fori_loop(0, NG, gbody_A, 0)
            _flush(qA)

            @pl.when(p + 1 < QI)
            def _next_a():
                pltpu.async_copy(
                    whp_hbm.at[pl.ds(2 * (qA + 2) * NPAD, 2 * NPAD)],
                    qwA, sem_a)
                pltpu.async_copy(
                    wdp_hbm.at[pl.ds(2 * (qA + 2) * NPAD, 2 * NPAD)],
                    qdA, sem_a)

            pre[2].wait()
            pre[3].wait()
            lax.fori_loop(0, NG, gbody_B, 0)
            _flush(qB)

            @pl.when(p + 1 < QI)
            def _next_b():
                pltpu.async_copy(
                    whp_hbm.at[pl.ds(2 * (qB + 2) * NPAD, 2 * NPAD)],
                    qwB, sem_b)
                pltpu.async_copy(
                    wdp_hbm.at[pl.ds(2 * (qB + 2) * NPAD, 2 * NPAD)],
                    qdB, sem_b)

            return carry

        lax.fori_loop(0, QI, pair_body, 0)

    return sc_kernel


_sc_kernel = _make_sc_kernel()


def kernel(h, W_ref, a_ref, W_dir, a_dir, ref_neighbors, dir_neighbors):
    h_pad = jnp.pad(h, ((0, NPAD - N), (0, 0)))
    Wc = jnp.concatenate([W_ref, W_dir], axis=1)
    a1 = a_ref[:OUT, 0]
    a2 = a_ref[OUT:, 0]
    b1 = a_dir[:OUT, 0]
    b2 = a_dir[OUT:, 0]
    z = jnp.zeros((OUT,), jnp.float32)
    Bmat = jnp.stack(
        [jnp.concatenate([a1, z]), jnp.concatenate([a2, z]),
         jnp.concatenate([z, b1]), jnp.concatenate([z, b2]),
         jnp.zeros((2 * OUT,), jnp.float32), jnp.zeros((2 * OUT,), jnp.float32),
         jnp.zeros((2 * OUT,), jnp.float32), jnp.zeros((2 * OUT,), jnp.float32)],
        axis=1)

    whT, whdT, st = _tc_project(h_pad, Wc, Bmat)

    ridx = jnp.pad(ref_neighbors.astype(jnp.int32),
                   ((0, NPAD - N), (0, 0), (0, 0))).reshape(-1)
    didx = jnp.pad(dir_neighbors.astype(jnp.int32),
                   ((0, NPAD - N), (0, 0))).reshape(-1)

    def _pack2(a, b):
        return lax.bitcast_convert_type(
            jnp.stack([a.astype(jnp.bfloat16), b.astype(jnp.bfloat16)],
                      axis=-1), jnp.int32)

    def _pack_pairs(t):
        return _pack2(t[0::2], t[1::2]).reshape(-1)

    tpk = _pack2(st[1], st[3])

    out_pad = _sc_kernel(_pack_pairs(whT), _pack_pairs(whdT),
                         st.reshape(8 * NPAD), tpk, ridx, didx)
    return jnp.transpose(out_pad.reshape(OUT, NPAD))[:N]


# speedup vs baseline: 1.6491x; 1.6491x over previous
import functools

import jax
import jax.numpy as jnp
from jax import lax
from jax.experimental import pallas as pl
from jax.experimental.pallas import tpu as pltpu
from jax.experimental.pallas import tpu_sc as plsc

N = 10000
NPAD = 10240
IN = 128
OUT = 32
D4 = 4
RK = 10
DK = 16
NEG = 0.2

NC = 2
NS = 16
NW = NC * NS
NT = NPAD // NW
NG = NT // 16
DCH = 32
NDCH = NT // DCH
RLEN = D4 * NT
NQ = OUT // 4
QI = NQ // 2


def _leaky(x):
    return jnp.where(x >= 0, x, NEG * x)


def _sigmoid(x):
    return 1.0 / (1.0 + jnp.exp(-x))


BN = 1024


def _pack_rows(lo, hi):
    u_lo = lax.bitcast_convert_type(lo.astype(jnp.bfloat16), jnp.uint16)
    u_hi = lax.bitcast_convert_type(hi.astype(jnp.bfloat16), jnp.uint16)
    return u_lo.astype(jnp.int32) | (u_hi.astype(jnp.int32) << 16)


def _tc_body(h_ref, wc_ref, bm_ref, whp_ref, wdp_ref, st_ref):
    PT = lax.dot_general(wc_ref[...], h_ref[...], (((0,), (1,)), ((), ())),
                         preferred_element_type=jnp.float32)
    whp_ref[...] = _pack_rows(PT[0:16], PT[16:32])
    wdp_ref[...] = _pack_rows(PT[32:48], PT[48:64])
    st_ref[...] = lax.dot_general(bm_ref[...], PT, (((0,), (0,)), ((), ())),
                                  preferred_element_type=jnp.float32)


def _tc_project(h_pad, Wc, Bmat):
    return pl.pallas_call(
        _tc_body,
        grid=(NPAD // BN,),
        in_specs=[
            pl.BlockSpec((BN, IN), lambda i: (i, 0)),
            pl.BlockSpec((IN, 2 * OUT), lambda i: (0, 0)),
            pl.BlockSpec((2 * OUT, 8), lambda i: (0, 0)),
        ],
        out_specs=[
            pl.BlockSpec((OUT // 2, BN), lambda i: (0, i)),
            pl.BlockSpec((OUT // 2, BN), lambda i: (0, i)),
            pl.BlockSpec((8, BN), lambda i: (0, i)),
        ],
        out_shape=[
            jax.ShapeDtypeStruct((OUT // 2, NPAD), jnp.int32),
            jax.ShapeDtypeStruct((OUT // 2, NPAD), jnp.int32),
            jax.ShapeDtypeStruct((8, NPAD), jnp.float32),
        ],
    )(h_pad, Wc, Bmat)


def _make_sc_kernel():
    mesh = plsc.VectorSubcoreMesh(core_axis_name="c", subcore_axis_name="s",
                                  num_cores=NC, num_subcores=NS)
    scratch = [
        pltpu.VMEM((NPAD,), jnp.int32),
        pltpu.VMEM((NT,), jnp.float32),
        pltpu.VMEM((NT,), jnp.float32),
        pltpu.VMEM((RK * RLEN,), jnp.int32),
        pltpu.VMEM((NT * DK,), jnp.int32),
        pltpu.VMEM((2 * NPAD,), jnp.int32),
        pltpu.VMEM((2 * NPAD,), jnp.int32),
        pltpu.VMEM((2 * NPAD,), jnp.int32),
        pltpu.VMEM((2 * NPAD,), jnp.int32),
        pltpu.VMEM((D4, NT), jnp.float32),
        pltpu.VMEM((DK, NT), jnp.float32),
        pltpu.VMEM((4 * NT,), jnp.float32),
        pltpu.SemaphoreType.DMA,
        pltpu.SemaphoreType.DMA,
    ]

    @functools.partial(
        pl.kernel,
        out_type=jax.ShapeDtypeStruct((OUT * NPAD,), jnp.float32),
        mesh=mesh,
        scratch_types=scratch,
        compiler_params=pltpu.CompilerParams(needs_layout_passes=False,
                                             use_tc_tiling_on_sc=False),
    )
    def sc_kernel(whp_hbm, wdp_hbm, st_hbm, tpk_hbm, ridx_hbm, didx_hbm,
                  out_hbm, tpk_v, s1_v, s2_v, ridx_v, didx_v,
                  qwA, qdA, qwB, qdB, alpha_v, ad_v, outq_v,
                  sem_a, sem_b):
        sid = lax.axis_index("s")
        wid = sid * NC + lax.axis_index("c")
        base = wid * NT

        def _unlo(w):
            return plsc.bitcast(w << 16, jnp.float32)

        def _unhi(w):
            return plsc.bitcast(w & jnp.int32(-65536), jnp.float32)

        pltpu.sync_copy(tpk_hbm, tpk_v)
        pltpu.sync_copy(st_hbm.at[pl.ds(base, NT)], s1_v)
        pltpu.sync_copy(st_hbm.at[pl.ds(2 * NPAD + base, NT)], s2_v)
        pltpu.sync_copy(ridx_hbm.at[pl.ds(wid * (RK * RLEN), RK * RLEN)],
                        ridx_v)
        pltpu.sync_copy(didx_hbm.at[pl.ds(wid * (NT * DK), NT * DK)], didx_v)

        pre = [pltpu.async_copy(whp_hbm.at[pl.ds(0, 2 * NPAD)], qwA, sem_a),
               pltpu.async_copy(wdp_hbm.at[pl.ds(0, 2 * NPAD)], qdA, sem_a),
               pltpu.async_copy(whp_hbm.at[pl.ds(2 * NPAD, 2 * NPAD)], qwB,
                                sem_b),
               pltpu.async_copy(wdp_hbm.at[pl.ds(2 * NPAD, 2 * NPAD)], qdB,
                                sem_b)]

        def attn_group(g, carry):
            goff = g * 16
            s1 = s1_v[pl.ds(goff, 16)]
            es = []
            for d in range(D4):
                acc = None
                for k in range(RK):
                    t = _unlo(plsc.load_gather(
                        tpk_v,
                        [ridx_v[pl.ds(k * RLEN + d * NT + goff, 16)]]))
                    acc = t if acc is None else acc + t
                es.append(_leaky(s1 + (1.0 / RK) * acc))
            m = jnp.maximum(jnp.maximum(es[0], es[1]),
                            jnp.maximum(es[2], es[3]))
            ex = [jnp.exp(e - m) for e in es]
            inv = (1.0 / RK) / ((ex[0] + ex[1]) + (ex[2] + ex[3]))
            for d in range(D4):
                alpha_v[d, pl.ds(goff, 16)] = ex[d] * inv

            s2 = s2_v[pl.ds(goff, 16)]
            doff = (g >> 1) * (DK * DCH) + (g & 1) * 16
            eds = []
            for k in range(DK):
                eds.append(_leaky(s2 + _unhi(plsc.load_gather(
                    tpk_v, [didx_v[pl.ds(doff + k * DCH, 16)]]))))
            m2 = functools.reduce(jnp.maximum, eds)
            ex2 = [jnp.exp(e - m2) for e in eds]
            inv2 = 1.0 / functools.reduce(lambda a, b: a + b, ex2)
            for k in range(DK):
                ad_v[k, pl.ds(goff, 16)] = ex2[k] * inv2
            return carry

        lax.fori_loop(0, NG, attn_group, 0)

        def make_quad_compute(col_wh, col_wd):
            def gbody(g, carry):
                goff = g * 16
                nloc = goff + lax.iota(jnp.int32, 16)
                als = [alpha_v[d, pl.ds(goff, 16)] for d in range(D4)]
                accR = [None] * 4
                for d in range(D4):
                    ts = [None] * 4
                    for k in range(RK):
                        iv = ridx_v[pl.ds(k * RLEN + d * NT + goff, 16)]
                        w0 = plsc.load_gather(col_wh, [iv])
                        w1 = plsc.load_gather(col_wh, [iv + NPAD])
                        for j, part in enumerate(
                                (_unlo(w0), _unhi(w0), _unlo(w1), _unhi(w1))):
                            ts[j] = part if ts[j] is None else ts[j] + part
                    for j in range(4):
                        wv = als[d] * ts[j]
                        accR[j] = wv if accR[j] is None else accR[j] + wv
                doff = (g >> 1) * (DK * DCH) + (g & 1) * 16
                accD = [None] * 4
                for k in range(DK):
                    iv = didx_v[pl.ds(doff + k * DCH, 16)]
                    w0 = plsc.load_gather(col_wd, [iv])
                    w1 = plsc.load_gather(col_wd, [iv + NPAD])
                    adk = ad_v[k, pl.ds(goff, 16)]
                    for j, part in enumerate(
                            (_unlo(w0), _unhi(w0), _unlo(w1), _unhi(w1))):
                        wv = adk * part
                        accD[j] = wv if accD[j] is None else accD[j] + wv
                for j in range(4):
                    val = 0.5 * (_sigmoid(accR[j]) + _sigmoid(accD[j]))
                    plsc.store_scatter(outq_v, [j * NT + nloc], val)
                return carry

            return gbody

        gbody_A = make_quad_compute(qwA, qdA)
        gbody_B = make_quad_compute(qwB, qdB)

        def _flush(q):
            for j in range(4):
                pltpu.sync_copy(
                    outq_v.at[pl.ds(j * NT, NT)],
                    out_hbm.at[pl.ds((4 * q + j) * NPAD + base, NT)])

        def pair_body(p, carry):
            qA = 2 * p
            qB = 2 * p + 1
            pre[0].wait()
            pre[1].wait()
            lax.fori_loop(0, NG, gbody_A, 0)
            _flush(qA)

            @pl.when(p + 1 < QI)
            def _next_a():
                pltpu.async_copy(
                    whp_hbm.at[pl.ds(2 * (qA + 2) * NPAD, 2 * NPAD)],
                    qwA, sem_a)
                pltpu.async_copy(
                    wdp_hbm.at[pl.ds(2 * (qA + 2) * NPAD, 2 * NPAD)],
                    qdA, sem_a)

            pre[2].wait()
            pre[3].wait()
            lax.fori_loop(0, NG, gbody_B, 0)
            _flush(qB)

            @pl.when(p + 1 < QI)
            def _next_b():
                pltpu.async_copy(
                    whp_hbm.at[pl.ds(2 * (qB + 2) * NPAD, 2 * NPAD)],
                    qwB, sem_b)
                pltpu.async_copy(
                    wdp_hbm.at[pl.ds(2 * (qB + 2) * NPAD, 2 * NPAD)],
                    qdB, sem_b)

            return carry

        lax.fori_loop(0, QI, pair_body, 0)

    return sc_kernel


_sc_kernel = _make_sc_kernel()


def kernel(h, W_ref, a_ref, W_dir, a_dir, ref_neighbors, dir_neighbors):
    h_pad = jnp.pad(h, ((0, NPAD - N), (0, 0)))
    Wc = jnp.concatenate([W_ref, W_dir], axis=1)
    a1 = a_ref[:OUT, 0]
    a2 = a_ref[OUT:, 0]
    b1 = a_dir[:OUT, 0]
    b2 = a_dir[OUT:, 0]
    z = jnp.zeros((OUT,), jnp.float32)
    Bmat = jnp.stack(
        [jnp.concatenate([a1, z]), jnp.concatenate([a2, z]),
         jnp.concatenate([z, b1]), jnp.concatenate([z, b2]),
         jnp.zeros((2 * OUT,), jnp.float32), jnp.zeros((2 * OUT,), jnp.float32),
         jnp.zeros((2 * OUT,), jnp.float32), jnp.zeros((2 * OUT,), jnp.float32)],
        axis=1)

    perm = [2 * i for i in range(16)] + [2 * i + 1 for i in range(16)]
    cperm = jnp.asarray(perm + [32 + p for p in perm], jnp.int32)
    whp, wdp, st = _tc_project(h_pad, Wc[:, cperm], Bmat[cperm])

    rn = jnp.pad(ref_neighbors.astype(jnp.int32),
                 ((0, NPAD - N), (0, 0), (0, 0)))
    ridx = rn.reshape(NW, NT, D4, RK).transpose(0, 3, 2, 1).reshape(-1)
    dn = jnp.pad(dir_neighbors.astype(jnp.int32), ((0, NPAD - N), (0, 0)))
    didx = dn.reshape(NW, NDCH, DCH, DK).transpose(0, 1, 3, 2).reshape(-1)

    tpk = lax.bitcast_convert_type(
        jnp.stack([st[1].astype(jnp.bfloat16), st[3].astype(jnp.bfloat16)],
                  axis=-1), jnp.int32)

    out_pad = _sc_kernel(whp.reshape(-1), wdp.reshape(-1),
                         st.reshape(8 * NPAD), tpk, ridx, didx)
    return jnp.transpose(out_pad.reshape(OUT, NPAD))[:N]
